# Initial kernel scaffold; baseline (speedup 1.0000x reference)
import jax
import jax.numpy as jnp
from jax.experimental import pallas as pl


def kernel(q_curve, u, taus):
    raise NotImplementedError("write your pallas kernel here")


# speedup vs baseline: 789.1051x; 789.1051x over previous
import functools

import jax
import jax.numpy as jnp
from jax import lax
from jax.experimental import pallas as pl
from jax.experimental.pallas import tpu as pltpu
from jax.experimental.pallas import tpu_sc as plsc

L = 16
NW = 32
NT = 4096 * 24
S = 128
CPW = NT // NW
GPW = CPW // L
SCHUNK = 4


def _sc_body(qT, u2, tsp, iv, out, qbuf, coef, tbuf, ibuf, ubuf, obuf):
    nc = 2
    wid = lax.axis_index("s") * nc + lax.axis_index("c")
    base = wid * CPW

    pltpu.sync_copy(qT.at[:, pl.ds(base, CPW)], qbuf)
    pltpu.sync_copy(tsp, tbuf)
    pltpu.sync_copy(iv, ibuf)

    ivecs = [ibuf[j, :] for j in range(8)]
    t0 = tbuf[0, :]
    tvecs = [tbuf[j, :] for j in range(1, 8)]

    @pl.loop(0, GPW)
    def _build(g):
        sl = pl.ds(g * L, L)
        cum = qbuf[0, sl]
        first = cum
        svecs = []
        for j in range(8):
            nxt = jnp.maximum(cum, qbuf[j + 1, sl])
            svecs.append((nxt - cum) * ivecs[j])
            cum = nxt
        coef[0, sl] = first - svecs[0] * t0
        coef[1, sl] = svecs[0]
        for j in range(1, 8):
            coef[1 + j, sl] = svecs[j] - svecs[j - 1]

    @pl.loop(0, S, step=SCHUNK)
    def _rows(s0):
        pltpu.sync_copy(u2.at[pl.ds(s0, SCHUNK), pl.ds(base, CPW)], ubuf)

        @pl.loop(0, GPW)
        def _grp(g):
            sl = pl.ds(g * L, L)
            cvecs = [coef[j, sl] for j in range(9)]
            for r in range(SCHUNK):
                uv = ubuf[r, sl]
                acc = cvecs[0] + cvecs[1] * uv
                for j in range(1, 8):
                    acc = acc + cvecs[1 + j] * jnp.maximum(uv - tvecs[j - 1], 0.0)
                obuf[r, sl] = jnp.maximum(acc, 0.0)

        pltpu.sync_copy(obuf, out.at[pl.ds(s0, SCHUNK), pl.ds(base, CPW)])


@jax.jit
def kernel(q_curve, u, taus):
    qT = q_curve.reshape(NT, 9).T
    u2 = u.reshape(S, NT)
    dt = taus[1:] - taus[:-1]
    ivs = 1.0 / (dt + 1e-12)
    tsp = jnp.broadcast_to(taus[:, None], (9, L)).astype(jnp.float32)
    ivb = jnp.broadcast_to(ivs[:, None], (8, L)).astype(jnp.float32)

    mesh = plsc.VectorSubcoreMesh(core_axis_name="c", subcore_axis_name="s")
    run = pl.kernel(
        _sc_body,
        out_type=jax.ShapeDtypeStruct((S, NT), jnp.float32),
        mesh=mesh,
        scratch_types=[
            pltpu.VMEM((9, CPW), jnp.float32),
            pltpu.VMEM((9, CPW), jnp.float32),
            pltpu.VMEM((9, L), jnp.float32),
            pltpu.VMEM((8, L), jnp.float32),
            pltpu.VMEM((SCHUNK, CPW), jnp.float32),
            pltpu.VMEM((SCHUNK, CPW), jnp.float32),
        ],
    )
    scen = run(qT, u2, tsp, ivb)
    return scen.reshape(S, 4096, 24)
